# baseline (speedup 1.0000x reference)
import jax
import jax.numpy as jnp
from jax import lax
from jax.experimental import pallas as pl
from jax.experimental.pallas import tpu as pltpu
from jax.experimental.pallas import tpu_sc as plsc

_ROWS = 32768
_COLS = 1024

_TC_ROWS = 22528
_SC_ROWS = _ROWS - _TC_ROWS

_BLK = 1024
_NSPLIT = 2
_PART = _TC_ROWS // _NSPLIT
_STEPS = _PART // _BLK

_NW = 32
_CHUNK_ROWS = 32
_SC_ROWS_PER_WORKER = _SC_ROWS // _NW
_NCHUNKS = _SC_ROWS_PER_WORKER // _CHUNK_ROWS


def _tc_body(*refs):
    x_refs = refs[:_NSPLIT]
    y_refs = refs[_NSPLIT : 2 * _NSPLIT]
    nx_ref, ny_ref = refs[2 * _NSPLIT], refs[2 * _NSPLIT + 1]
    accx, accy = refs[2 * _NSPLIT + 2], refs[2 * _NSPLIT + 3]
    i = pl.program_id(0)

    @pl.when(i == 0)
    def _init():
        accx[...] = jnp.zeros_like(accx)
        accy[...] = jnp.zeros_like(accy)

    def csum(ref):
        s = (ref[...] > 0).astype(jnp.int32).reshape(_BLK // 8, 8, _COLS)
        return jnp.sum(s, axis=0)

    ax = csum(x_refs[0])
    ay = csum(y_refs[0])
    for k in range(1, _NSPLIT):
        ax = ax + csum(x_refs[k])
        ay = ay + csum(y_refs[k])
    accx[...] += ax
    accy[...] += ay

    @pl.when(i == _STEPS - 1)
    def _fin():
        nx_ref[0, 0] = jnp.sum(accx[...])
        ny_ref[0, 0] = jnp.sum(accy[...])


def _count_chunk(buf, slot, acc):
    one = jnp.ones((16,), jnp.int32)
    zero = jnp.zeros((16,), jnp.int32)

    def body(i, acc):
        r = i // 4
        cb = (i % 4) * 256
        for u in range(16):
            v = buf[slot, r, pl.ds(cb + u * 16, 16)]
            acc = acc + jnp.where(v > 0, one, zero)
        return acc

    return lax.fori_loop(0, _CHUNK_ROWS * 4, body, acc)


def _sc_body(x_ref, y_ref, out_ref, buf, accv, sem0, sem1):
    wid = lax.axis_index("s") * 2 + lax.axis_index("c")
    row0 = _TC_ROWS + wid * _SC_ROWS_PER_WORKER
    sems = (sem0, sem1)

    def start(arr_ref, c, slot):
        pltpu.async_copy(
            arr_ref.at[pl.ds(row0 + c * _CHUNK_ROWS, _CHUNK_ROWS), :],
            buf.at[slot],
            sems[slot],
        )

    def wait(arr_ref, c, slot):
        pltpu.make_async_copy(
            arr_ref.at[pl.ds(row0 + c * _CHUNK_ROWS, _CHUNK_ROWS), :],
            buf.at[slot],
            sems[slot],
        ).wait()

    def count_array(arr_ref, arr_idx):
        start(arr_ref, 0, 0)
        start(arr_ref, 1, 1)

        def body(g, acc):
            for slot in range(2):
                c = g * 2 + slot
                wait(arr_ref, c, slot)
                acc = _count_chunk(buf, slot, acc)

                @pl.when(c + 2 < _NCHUNKS)
                def _():
                    start(arr_ref, c + 2, slot)

            return acc

        acc = lax.fori_loop(0, _NCHUNKS // 2, body, jnp.zeros((16,), jnp.int32))
        accv[...] = acc
        pltpu.sync_copy(accv, out_ref.at[pl.ds(arr_idx * _NW * 16 + wid * 16, 16)])

    count_array(x_ref, 0)
    count_array(y_ref, 1)


def kernel(x, y):
    def part(k):
        return pl.BlockSpec((_BLK, _COLS), lambda i, k=k: (i + k * _STEPS, 0))

    specs = [part(k) for k in range(_NSPLIT)]
    nx_tc, ny_tc = pl.pallas_call(
        _tc_body,
        grid=(_STEPS,),
        in_specs=specs + specs,
        out_specs=[
            pl.BlockSpec(memory_space=pltpu.SMEM),
            pl.BlockSpec(memory_space=pltpu.SMEM),
        ],
        out_shape=[
            jax.ShapeDtypeStruct((1, 1), jnp.int32),
            jax.ShapeDtypeStruct((1, 1), jnp.int32),
        ],
        scratch_shapes=[
            pltpu.VMEM((8, _COLS), jnp.int32),
            pltpu.VMEM((8, _COLS), jnp.int32),
        ],
    )(x, x, y, y)

    mesh = plsc.VectorSubcoreMesh(core_axis_name="c", subcore_axis_name="s")
    sc_k = pl.kernel(
        _sc_body,
        out_type=jax.ShapeDtypeStruct((2 * _NW * 16,), jnp.int32),
        mesh=mesh,
        scratch_types=[
            pltpu.VMEM((2, _CHUNK_ROWS, _COLS), jnp.float32),
            pltpu.VMEM((16,), jnp.int32),
            pltpu.SemaphoreType.DMA,
            pltpu.SemaphoreType.DMA,
        ],
        compiler_params=pltpu.CompilerParams(use_tc_tiling_on_sc=True),
    )
    sc_partials = sc_k(x, y)

    sc_counts = sc_partials.reshape(2, _NW * 16).sum(axis=1)
    return jnp.maximum(
        nx_tc[0, 0] + sc_counts[0], ny_tc[0, 0] + sc_counts[1]
    )


# speedup vs baseline: 1.2579x; 1.2579x over previous
import jax
import jax.numpy as jnp
from jax.experimental import pallas as pl
from jax.experimental.pallas import tpu as pltpu

_ROWS = 32768
_COLS = 1024
_BLK = 1024
_NSPLIT = 2
_PART = _ROWS // _NSPLIT
_STEPS = _PART // _BLK


def _tc_body(*refs):
    x_refs = refs[:_NSPLIT]
    y_refs = refs[_NSPLIT : 2 * _NSPLIT]
    nx_ref, ny_ref = refs[2 * _NSPLIT], refs[2 * _NSPLIT + 1]
    accx, accy = refs[2 * _NSPLIT + 2], refs[2 * _NSPLIT + 3]
    i = pl.program_id(0)

    @pl.when(i == 0)
    def _init():
        accx[...] = jnp.zeros_like(accx)
        accy[...] = jnp.zeros_like(accy)

    def csum(ref):
        s = (ref[...] > 0).astype(jnp.int32).reshape(_BLK // 8, 8, _COLS)
        return jnp.sum(s, axis=0)

    ax = csum(x_refs[0])
    ay = csum(y_refs[0])
    for k in range(1, _NSPLIT):
        ax = ax + csum(x_refs[k])
        ay = ay + csum(y_refs[k])
    accx[...] += ax
    accy[...] += ay

    @pl.when(i == _STEPS - 1)
    def _fin():
        nx_ref[0, 0] = jnp.sum(accx[...])
        ny_ref[0, 0] = jnp.sum(accy[...])


def kernel(x, y):
    def part(k):
        return pl.BlockSpec((_BLK, _COLS), lambda i, k=k: (i + k * _STEPS, 0))

    specs = [part(k) for k in range(_NSPLIT)]
    nx, ny = pl.pallas_call(
        _tc_body,
        grid=(_STEPS,),
        in_specs=specs + specs,
        out_specs=[
            pl.BlockSpec(memory_space=pltpu.SMEM),
            pl.BlockSpec(memory_space=pltpu.SMEM),
        ],
        out_shape=[
            jax.ShapeDtypeStruct((1, 1), jnp.int32),
            jax.ShapeDtypeStruct((1, 1), jnp.int32),
        ],
        scratch_shapes=[
            pltpu.VMEM((8, _COLS), jnp.int32),
            pltpu.VMEM((8, _COLS), jnp.int32),
        ],
    )(*([x] * _NSPLIT + [y] * _NSPLIT))
    return jnp.maximum(nx[0, 0], ny[0, 0])
